# Initial kernel scaffold; baseline (speedup 1.0000x reference)
import jax
import jax.numpy as jnp
from jax.experimental import pallas as pl


def kernel(x, edge_index, W1, b1, W2, b2, W3, b3):
    raise NotImplementedError("write your pallas kernel here")


# speedup vs baseline: 16.9115x; 16.9115x over previous
import functools

import jax
import jax.numpy as jnp
from jax import lax
from jax.experimental import pallas as pl
from jax.experimental.pallas import tpu as pltpu
from jax.experimental.pallas import tpu_sc as plsc

N_NODES = 10000
NPAD = 10240
D = 128
E_TOTAL = 320000

NC = 2
NS = 16
NW = NC * NS
EB = 80
BLOCKS = E_TOTAL // (NW * EB)
ROWS_PER_TILE = NPAD // NS
DEG_W = 16

def _zero_vmem(ref, rows, width):
    z = jnp.zeros((16,), jnp.float32)

    def body(i, _):
        for j in range(width // 16):
            ref[i, pl.ds(j * 16, 16)] = z
        return 0

    lax.fori_loop(0, rows, body, 0)


def _deg_body(dst_hbm, out_hbm, dst_v, ones_v, zero_v, acc):
    c = lax.axis_index("c")
    s = lax.axis_index("s")
    w = c * NS + s

    one = jnp.full((16,), 1.0, jnp.float32)

    def fill(i, _):
        ones_v[i, pl.ds(0, 16)] = one
        zero_v[i, pl.ds(0, 16)] = jnp.zeros((16,), jnp.float32)
        return 0

    lax.fori_loop(0, EB, fill, 0)

    for r in range(ROWS_PER_TILE // EB):
        pltpu.sync_copy(zero_v, acc.at[pl.ds(s * ROWS_PER_TILE + r * EB, EB)])
    plsc.subcore_barrier()

    pltpu.sync_copy(dst_hbm.at[w], dst_v)

    def body(k, _):
        pltpu.sync_copy(ones_v, acc.at[dst_v.at[k]], add=True)
        return 0

    lax.fori_loop(0, BLOCKS, body, 0)
    plsc.subcore_barrier()

    pltpu.sync_copy(acc.at[pl.ds(s * ROWS_PER_TILE, ROWS_PER_TILE)],
                    out_hbm.at[c, pl.ds(s * ROWS_PER_TILE, ROWS_PER_TILE)])


def _agg_body(h2_hbm, src_hbm, dst_hbm, out_hbm,
              src_v, dst_v, rows_v, acc, sem):
    c = lax.axis_index("c")
    s = lax.axis_index("s")
    w = c * NS + s

    _zero_vmem(rows_v, EB, D)
    for r in range(ROWS_PER_TILE // EB):
        pltpu.sync_copy(rows_v, acc.at[pl.ds(s * ROWS_PER_TILE + r * EB, EB)])
    plsc.subcore_barrier()

    pltpu.sync_copy(src_hbm.at[w], src_v)
    pltpu.sync_copy(dst_hbm.at[w], dst_v)

    def body(k, _):
        pltpu.async_copy(h2_hbm.at[src_v.at[k]], rows_v, sem).wait()
        pltpu.sync_copy(rows_v, acc.at[dst_v.at[k]], add=True)
        return 0

    lax.fori_loop(0, BLOCKS, body, 0)
    plsc.subcore_barrier()

    pltpu.sync_copy(acc.at[pl.ds(s * ROWS_PER_TILE, ROWS_PER_TILE)],
                    out_hbm.at[c, pl.ds(s * ROWS_PER_TILE, ROWS_PER_TILE)])


@functools.cache
def _sc_kernels():
    mesh = plsc.VectorSubcoreMesh(core_axis_name="c", subcore_axis_name="s",
                                  num_cores=NC, num_subcores=NS)
    deg = pl.kernel(
        _deg_body,
        out_type=jax.ShapeDtypeStruct((NC, NPAD, DEG_W), jnp.float32),
        mesh=mesh,
        scratch_types=[
            pltpu.VMEM((BLOCKS, EB), jnp.int32),
            pltpu.VMEM((EB, DEG_W), jnp.float32),
            pltpu.VMEM((EB, DEG_W), jnp.float32),
            pltpu.VMEM_SHARED((NPAD, DEG_W), jnp.float32),
        ],
    )
    agg = pl.kernel(
        _agg_body,
        out_type=jax.ShapeDtypeStruct((NC, NPAD, D), jnp.float32),
        mesh=mesh,
        scratch_types=[
            pltpu.VMEM((BLOCKS, EB), jnp.int32),
            pltpu.VMEM((BLOCKS, EB), jnp.int32),
            pltpu.VMEM((EB, D), jnp.float32),
            pltpu.VMEM_SHARED((NPAD, D), jnp.float32),
            pltpu.SemaphoreType.DMA,
        ],
    )
    return deg, agg


RB = 2048


def _dinv_from_parts(dp):
    deg = dp[0, :, 0] + dp[1, :, 0] + 1.0
    return lax.rsqrt(deg)


def _first_body(x_ref, w_ref, dp_ref, h2_ref):
    dinv = _dinv_from_parts(dp_ref[...])
    h = jnp.dot(x_ref[...], w_ref[...], preferred_element_type=jnp.float32)
    h2_ref[...] = h * dinv[:, None]


def _mid_body(agg_ref, h2_ref, dp_ref, b_ref, w_ref, out_ref):
    dinv = _dinv_from_parts(dp_ref[...])
    tot = agg_ref[0] + agg_ref[1] + h2_ref[...]
    hn = jnp.maximum(tot * dinv[:, None] + b_ref[...], 0.0)
    out_ref[...] = jnp.dot(hn, w_ref[...],
                           preferred_element_type=jnp.float32) * dinv[:, None]


def _last_body(agg_ref, h2_ref, dp_ref, b_ref, out_ref):
    dinv = _dinv_from_parts(dp_ref[...])
    tot = agg_ref[0] + agg_ref[1] + h2_ref[...]
    out_ref[...] = tot * dinv[:, None] + b_ref[...]


def _row_spec(width):
    return pl.BlockSpec((RB, width), lambda i: (i, 0))


def _part_spec(width):
    return pl.BlockSpec((NC, RB, width), lambda i: (0, i, 0))


_FULL_W = pl.BlockSpec((D, D), lambda i: (0, 0))
_FULL_B = pl.BlockSpec((1, D), lambda i: (0, 0))
_GRID = (NPAD // RB,)
_OUT = jax.ShapeDtypeStruct((NPAD, D), jnp.float32)

_first_call = pl.pallas_call(
    _first_body, grid=_GRID,
    in_specs=[_row_spec(D), _FULL_W, _part_spec(DEG_W)],
    out_specs=_row_spec(D), out_shape=_OUT)

_mid_call = pl.pallas_call(
    _mid_body, grid=_GRID,
    in_specs=[_part_spec(D), _row_spec(D), _part_spec(DEG_W), _FULL_B, _FULL_W],
    out_specs=_row_spec(D), out_shape=_OUT)

_last_call = pl.pallas_call(
    _last_body, grid=_GRID,
    in_specs=[_part_spec(D), _row_spec(D), _part_spec(DEG_W), _FULL_B],
    out_specs=_row_spec(D), out_shape=_OUT)


@jax.jit
def kernel(x, edge_index, W1, b1, W2, b2, W3, b3):
    src2d = edge_index[0].reshape(NW, BLOCKS, EB)
    dst2d = edge_index[1].reshape(NW, BLOCKS, EB)
    x_pad = jnp.pad(x, ((0, NPAD - N_NODES), (0, 0)))
    b1r = b1.reshape(1, D)
    b2r = b2.reshape(1, D)
    b3r = b3.reshape(1, D)

    deg_kernel, agg_kernel = _sc_kernels()
    deg_parts = deg_kernel(dst2d)

    h2 = _first_call(x_pad, W1, deg_parts)
    agg = agg_kernel(h2, src2d, dst2d)
    h2 = _mid_call(agg, h2, deg_parts, b1r, W2)
    agg = agg_kernel(h2, src2d, dst2d)
    h2 = _mid_call(agg, h2, deg_parts, b2r, W3)
    agg = agg_kernel(h2, src2d, dst2d)
    out = _last_call(agg, h2, deg_parts, b3r)
    return out[:N_NODES]


# speedup vs baseline: 25.9830x; 1.5364x over previous
import functools

import jax
import jax.numpy as jnp
from jax import lax
from jax.experimental import pallas as pl
from jax.experimental.pallas import tpu as pltpu
from jax.experimental.pallas import tpu_sc as plsc

N_NODES = 10000
NPAD = 10240
D = 128
E_TOTAL = 320000

NC = 2
NS = 16
NW = NC * NS
EB = 80
BLOCKS = E_TOTAL // (NW * EB)
ROWS_PER_TILE = NPAD // NS
DEG_W = 16


def _zero_vmem(ref, rows, width):
    z = jnp.zeros((16,), jnp.float32)

    def body(i, _):
        for j in range(width // 16):
            ref[i, pl.ds(j * 16, 16)] = z
        return 0

    lax.fori_loop(0, rows, body, 0)


def _deg_body(dst_hbm, out_hbm, dst_v, ones_v, zero_v, acc):
    c = lax.axis_index("c")
    s = lax.axis_index("s")
    w = c * NS + s

    one = jnp.full((16,), 1.0, jnp.float32)

    def fill(i, _):
        ones_v[i, pl.ds(0, 16)] = one
        zero_v[i, pl.ds(0, 16)] = jnp.zeros((16,), jnp.float32)
        return 0

    lax.fori_loop(0, EB, fill, 0)

    for r in range(ROWS_PER_TILE // EB):
        pltpu.sync_copy(zero_v, acc.at[pl.ds(s * ROWS_PER_TILE + r * EB, EB)])
    plsc.subcore_barrier()

    pltpu.sync_copy(dst_hbm.at[w], dst_v)

    def body(k, _):
        pltpu.sync_copy(ones_v, acc.at[dst_v.at[k]], add=True)
        return 0

    lax.fori_loop(0, BLOCKS, body, 0)
    plsc.subcore_barrier()

    pltpu.sync_copy(acc.at[pl.ds(s * ROWS_PER_TILE, ROWS_PER_TILE)],
                    out_hbm.at[c, pl.ds(s * ROWS_PER_TILE, ROWS_PER_TILE)])


CH = 5
N_CH = BLOCKS // CH
UNIT = 2 * CH
T_ITERS = (BLOCKS - CH) // UNIT


def _agg_body(h2_hbm, src_hbm, dst_hbm, out_hbm, sa, sb, da, db,
              rows0, rows1, acc, sem_g, sem_s, sem_d):
    c = lax.axis_index("c")
    s = lax.axis_index("s")
    w = c * NS + s

    _zero_vmem(rows0, EB, D)
    for r in range(ROWS_PER_TILE // EB):
        pltpu.sync_copy(rows0, acc.at[pl.ds(s * ROWS_PER_TILE + r * EB, EB)])
    plsc.subcore_barrier()

    rows = (rows0, rows1)
    last = jnp.int32(N_CH - 1)

    def s_load(j, buf):
        pltpu.async_copy(src_hbm.at[w, j], buf, sem_s)

    def s_wait(j, buf):
        pltpu.make_async_copy(src_hbm.at[w, j], buf, sem_s).wait()

    def d_load(j, buf):
        pltpu.async_copy(dst_hbm.at[w, j], buf, sem_d)

    def d_wait(j, buf):
        pltpu.make_async_copy(dst_hbm.at[w, j], buf, sem_d).wait()

    def gather(sbuf, b_loc, rbuf):
        pltpu.async_copy(h2_hbm.at[sbuf.at[b_loc]], rbuf, sem_g)

    def g_wait(sbuf, b_loc, rbuf):
        pltpu.make_async_copy(h2_hbm.at[sbuf.at[b_loc]], rbuf, sem_g).wait()

    def scatter(dbuf, b_loc, rbuf):
        pltpu.sync_copy(rbuf, acc.at[dbuf.at[b_loc]], add=True)

    pltpu.sync_copy(src_hbm.at[w, 0], sa)
    d_load(0, da)
    s_load(1, sb)
    d_load(1, db)
    gather(sa, 0, rows0)

    def body(t, _):
        j = 2 * t
        d_wait(j, da)
        for i in range(UNIT):
            cur_s, cur_d, b = (sa, da, i) if i < CH else (sb, db, i - CH)
            if i == CH:
                d_wait(j + 1, db)
            if i < UNIT - 1:
                if b + 1 < CH:
                    gather(cur_s, b + 1, rows[(i + 1) % 2])
                else:
                    s_wait(j + 1, sb)
                    gather(sb, 0, rows[(i + 1) % 2])
            else:
                s_wait(jnp.minimum(j + 2, last), sa)
                gather(sa, 0, rows[(i + 1) % 2])
            g_wait(cur_s, b, rows[i % 2])
            scatter(cur_d, b, rows[i % 2])
            if i == CH:
                s_load(jnp.minimum(j + 2, last), sa)
            if i == UNIT - 1:
                d_load(jnp.minimum(j + 2, last), da)
                s_load(jnp.minimum(j + 3, last), sb)
                d_load(jnp.minimum(j + 3, last), db)
        return 0

    lax.fori_loop(0, T_ITERS, body, 0)

    d_wait(last, da)
    for i in range(CH):
        gather(sa, min(i + 1, CH - 1), rows[(i + 1) % 2])
        g_wait(sa, i, rows[i % 2])
        scatter(da, i, rows[i % 2])
    g_wait(sa, CH - 1, rows[CH % 2])
    s_wait(last, sb)
    d_wait(last, db)
    plsc.subcore_barrier()

    pltpu.sync_copy(acc.at[pl.ds(s * ROWS_PER_TILE, ROWS_PER_TILE)],
                    out_hbm.at[c, pl.ds(s * ROWS_PER_TILE, ROWS_PER_TILE)])


@functools.cache
def _sc_kernels():
    mesh = plsc.VectorSubcoreMesh(core_axis_name="c", subcore_axis_name="s",
                                  num_cores=NC, num_subcores=NS)
    deg = pl.kernel(
        _deg_body,
        out_type=jax.ShapeDtypeStruct((NC, NPAD, DEG_W), jnp.float32),
        mesh=mesh,
        scratch_types=[
            pltpu.VMEM((BLOCKS, EB), jnp.int32),
            pltpu.VMEM((EB, DEG_W), jnp.float32),
            pltpu.VMEM((EB, DEG_W), jnp.float32),
            pltpu.VMEM_SHARED((NPAD, DEG_W), jnp.float32),
        ],
    )
    agg = pl.kernel(
        _agg_body,
        out_type=jax.ShapeDtypeStruct((NC, NPAD, D), jnp.float32),
        mesh=mesh,
        scratch_types=[
            pltpu.VMEM((CH, EB), jnp.int32),
            pltpu.VMEM((CH, EB), jnp.int32),
            pltpu.VMEM((CH, EB), jnp.int32),
            pltpu.VMEM((CH, EB), jnp.int32),
            pltpu.VMEM((EB, D), jnp.float32),
            pltpu.VMEM((EB, D), jnp.float32),
            pltpu.VMEM_SHARED((NPAD, D), jnp.float32),
            pltpu.SemaphoreType.DMA,
            pltpu.SemaphoreType.DMA,
            pltpu.SemaphoreType.DMA,
        ],
    )
    return deg, agg


RB = 2048


def _dinv_from_parts(dp):
    deg = dp[0, :, 0] + dp[1, :, 0] + 1.0
    return lax.rsqrt(deg)


def _first_body(x_ref, w_ref, dp_ref, h2_ref):
    dinv = _dinv_from_parts(dp_ref[...])
    h = jnp.dot(x_ref[...], w_ref[...], preferred_element_type=jnp.float32)
    h2_ref[...] = h * dinv[:, None]


def _mid_body(agg_ref, h2_ref, dp_ref, b_ref, w_ref, out_ref):
    dinv = _dinv_from_parts(dp_ref[...])
    tot = agg_ref[0] + agg_ref[1] + h2_ref[...]
    hn = jnp.maximum(tot * dinv[:, None] + b_ref[...], 0.0)
    out_ref[...] = jnp.dot(hn, w_ref[...],
                           preferred_element_type=jnp.float32) * dinv[:, None]


def _last_body(agg_ref, h2_ref, dp_ref, b_ref, out_ref):
    dinv = _dinv_from_parts(dp_ref[...])
    tot = agg_ref[0] + agg_ref[1] + h2_ref[...]
    out_ref[...] = tot * dinv[:, None] + b_ref[...]


def _row_spec(width):
    return pl.BlockSpec((RB, width), lambda i: (i, 0))


def _part_spec(width):
    return pl.BlockSpec((NC, RB, width), lambda i: (0, i, 0))


_FULL_W = pl.BlockSpec((D, D), lambda i: (0, 0))
_FULL_B = pl.BlockSpec((1, D), lambda i: (0, 0))
_GRID = (NPAD // RB,)
_OUT = jax.ShapeDtypeStruct((NPAD, D), jnp.float32)

_first_call = pl.pallas_call(
    _first_body, grid=_GRID,
    in_specs=[_row_spec(D), _FULL_W, _part_spec(DEG_W)],
    out_specs=_row_spec(D), out_shape=_OUT)

_mid_call = pl.pallas_call(
    _mid_body, grid=_GRID,
    in_specs=[_part_spec(D), _row_spec(D), _part_spec(DEG_W), _FULL_B,
              _FULL_W],
    out_specs=_row_spec(D), out_shape=_OUT)

_last_call = pl.pallas_call(
    _last_body, grid=_GRID,
    in_specs=[_part_spec(D), _row_spec(D), _part_spec(DEG_W), _FULL_B],
    out_specs=_row_spec(D), out_shape=_OUT)


@jax.jit
def kernel(x, edge_index, W1, b1, W2, b2, W3, b3):
    src4 = edge_index[0].reshape(NW, N_CH, CH, EB)
    dst4 = edge_index[1].reshape(NW, N_CH, CH, EB)
    dst2d = edge_index[1].reshape(NW, BLOCKS, EB)
    x_pad = jnp.pad(x, ((0, NPAD - N_NODES), (0, 0)))
    b1r = b1.reshape(1, D)
    b2r = b2.reshape(1, D)
    b3r = b3.reshape(1, D)

    deg_kernel, agg_kernel = _sc_kernels()
    deg_parts = deg_kernel(dst2d)

    h2 = _first_call(x_pad, W1, deg_parts)
    agg = agg_kernel(h2, src4, dst4)
    h2 = _mid_call(agg, h2, deg_parts, b1r, W2)
    agg = agg_kernel(h2, src4, dst4)
    h2 = _mid_call(agg, h2, deg_parts, b2r, W3)
    agg = agg_kernel(h2, src4, dst4)
    out = _last_call(agg, h2, deg_parts, b3r)
    return out[:N_NODES]
